# baseline (speedup 1.0000x reference)
import functools

import jax
import jax.numpy as jnp
from jax import lax
from jax.experimental import pallas as pl
from jax.experimental.pallas import tpu as pltpu
from jax.experimental.pallas import tpu_sc as plsc

B, H, N_SMALL, N, D = 8, 4, 1024, 2048, 128
CA = 240
W = 256
CHUNKS = [(i * CA, CA) for i in range(8)] + [(8 * CA, N - 8 * CA)]
THRESH = [n0 for n0, _ in CHUNKS[1:]]
LANES = 16


NB = 8
CROWS = 512


def _copy_body(src_ref, dst_ref, *scratch):
    bufs = scratch[:NB]
    isems = scratch[NB:2 * NB]
    osems = scratch[2 * NB:]
    nchunks = src_ref.shape[0] // CROWS
    cin = [None] * NB
    cout = [None] * NB
    for i in range(NB):
        cin[i] = pltpu.async_copy(
            src_ref.at[pl.ds(i * CROWS, CROWS)], bufs[i], isems[i])
    for i in range(nchunks):
        bi = i % NB
        cin[bi].wait()
        cout[bi] = pltpu.async_copy(
            bufs[bi], dst_ref.at[pl.ds(i * CROWS, CROWS)], osems[bi])
        j = i + NB
        if j < nchunks:
            cout[bi].wait()
            cin[bi] = pltpu.async_copy(
                src_ref.at[pl.ds(j * CROWS, CROWS)], bufs[bi], isems[bi])
    for i in range(max(0, nchunks - NB), nchunks):
        cout[i % NB].wait()


def _tc_copy(x2d, block_rows):
    del block_rows
    rows, cols = x2d.shape
    return pl.pallas_call(
        _copy_body,
        in_specs=[pl.BlockSpec(memory_space=pltpu.MemorySpace.HBM)],
        out_specs=pl.BlockSpec(memory_space=pltpu.MemorySpace.HBM),
        out_shape=jax.ShapeDtypeStruct((rows, cols), x2d.dtype),
        scratch_shapes=(
            [pltpu.VMEM((CROWS, 2048), jnp.float32)] * NB
            + [pltpu.SemaphoreType.DMA] * (2 * NB)
        ),
    )(x2d)


def _sc_unpool(h, pre_h, idx32):
    mesh = plsc.VectorSubcoreMesh(core_axis_name="c", subcore_axis_name="s")

    @functools.partial(
        pl.kernel,
        mesh=mesh,
        out_type=jax.ShapeDtypeStruct((B, H, N, D), jnp.float32),
        scratch_types=[
            pltpu.VMEM((CA, D), jnp.float32),
            pltpu.VMEM((CA, D), jnp.float32),
            pltpu.VMEM((W, D), jnp.float32),
            pltpu.VMEM((W, D), jnp.float32),
            pltpu.VMEM((N_SMALL + LANES,), jnp.int32),
        ] + [pltpu.SemaphoreType.DMA] * 6,
    )
    def k(h_hbm, pre_hbm, idx_hbm, out_hbm, ob0, ob1, hb0, hb1, idxv,
          sp0, sp1, sh0, sh1, sw0, sw1):
        cid = lax.axis_index("c")
        sid = lax.axis_index("s")
        wid = sid * 2 + cid
        b = wid // H
        hh = wid % H

        pltpu.sync_copy(idx_hbm.at[b], idxv.at[pl.ds(0, N_SMALL)])

        def searchsorted(t):
            def bs_body(step, lo):
                probe = lo + jnp.int32(2 ** (9 - step))
                v = idxv[pl.ds(probe - 1, LANES)][0]
                return jnp.where((probe <= N_SMALL) & (v < t), probe, lo)
            return lax.fori_loop(0, 10, bs_body, jnp.int32(0), unroll=True)
        js = [jnp.int32(0)] + [searchsorted(jnp.int32(t)) for t in THRESH]
        j0a = [(jnp.minimum(j, jnp.int32(N_SMALL - W)) // 8) * 8
               for j in js]

        obufs = (ob0, ob1)
        hbufs = (hb0, hb1)
        psems = (sp0, sp1)
        hsems = (sh0, sh1)
        wsems = (sw0, sw1)
        p_cps = [None, None]
        h_cps = [None, None]
        w_cps = [None, None]

        def start_reads(c):
            bi = c & 1
            n0, cc = CHUNKS[c]
            p_cps[bi] = pltpu.async_copy(
                pre_hbm.at[b, hh, pl.ds(n0, cc)],
                obufs[bi].at[pl.ds(0, cc)], psems[bi])
            h_cps[bi] = pltpu.async_copy(
                h_hbm.at[b, hh, pl.ds(j0a[c], W)], hbufs[bi], hsems[bi])

        start_reads(0)
        for c in range(len(CHUNKS)):
            bi = c & 1
            n0, cc = CHUNKS[c]
            p_cps[bi].wait()
            h_cps[bi].wait()
            if c + 1 < len(CHUNKS):
                ob = (c + 1) & 1
                if w_cps[ob] is not None:
                    w_cps[ob].wait()
                start_reads(c + 1)

            obuf = obufs[bi]
            hbuf = hbufs[bi]
            base = j0a[c]
            je = js[c + 1] if c + 1 < len(CHUNKS) else jnp.int32(N_SMALL)

            def place(j, carry):
                t = idxv[pl.ds(j, LANES)][0] - n0
                hr = j - base
                for l in range(D // LANES):
                    sl = pl.ds(l * LANES, LANES)
                    plsc.addupdate(obuf.at[t, sl], hbuf[hr, sl])
                return carry
            lax.fori_loop(js[c], je, place, 0, unroll=False)

            w_cps[bi] = pltpu.async_copy(
                obuf.at[pl.ds(0, cc)],
                out_hbm.at[b, hh, pl.ds(n0, cc)], wsems[bi])
        w_cps[0].wait()
        w_cps[1].wait()

    return k(h, pre_h, idx32)


def kernel(g, h, pre_h, idx):
    idx32 = idx.astype(jnp.int32)
    new_h = _sc_unpool(h, pre_h, idx32)
    g_out = _tc_copy(g.reshape(B * N, N), 1024).reshape(B, N, N)
    return (g_out, new_h)


# speedup vs baseline: 1.0012x; 1.0012x over previous
import functools

import jax
import jax.numpy as jnp
from jax import lax
from jax.experimental import pallas as pl
from jax.experimental.pallas import tpu as pltpu
from jax.experimental.pallas import tpu_sc as plsc

B, H, N_SMALL, N, D = 8, 4, 1024, 2048, 128
CA = 240
W = 256
CHUNKS = [(i * CA, CA) for i in range(8)] + [(8 * CA, N - 8 * CA)]
THRESH = [n0 for n0, _ in CHUNKS[1:]]
LANES = 16


NB = 4
CROWS = 512


def _copy_body(src_ref, dst_ref, *scratch):
    bufs = scratch[:NB]
    isems = scratch[NB:2 * NB]
    osems = scratch[2 * NB:]
    nchunks = src_ref.shape[0] // CROWS
    cin = [None] * NB
    cout = [None] * NB
    for i in range(NB):
        cin[i] = pltpu.async_copy(
            src_ref.at[pl.ds(i * CROWS, CROWS)], bufs[i], isems[i])
    for i in range(nchunks):
        bi = i % NB
        cin[bi].wait()
        cout[bi] = pltpu.async_copy(
            bufs[bi], dst_ref.at[pl.ds(i * CROWS, CROWS)], osems[bi])
        j = i + NB
        if j < nchunks:
            cout[bi].wait()
            cin[bi] = pltpu.async_copy(
                src_ref.at[pl.ds(j * CROWS, CROWS)], bufs[bi], isems[bi])
    for i in range(max(0, nchunks - NB), nchunks):
        cout[i % NB].wait()


def _tc_copy(x2d, block_rows):
    del block_rows
    rows, cols = x2d.shape
    return pl.pallas_call(
        _copy_body,
        in_specs=[pl.BlockSpec(memory_space=pltpu.MemorySpace.HBM)],
        out_specs=pl.BlockSpec(memory_space=pltpu.MemorySpace.HBM),
        out_shape=jax.ShapeDtypeStruct((rows, cols), x2d.dtype),
        scratch_shapes=(
            [pltpu.VMEM((CROWS, 2048), jnp.float32)] * NB
            + [pltpu.SemaphoreType.DMA] * (2 * NB)
        ),
    )(x2d)


def _sc_unpool(h, pre_h, idx32):
    mesh = plsc.VectorSubcoreMesh(core_axis_name="c", subcore_axis_name="s")

    @functools.partial(
        pl.kernel,
        mesh=mesh,
        out_type=jax.ShapeDtypeStruct((B, H, N, D), jnp.float32),
        scratch_types=[
            pltpu.VMEM((CA, D), jnp.float32),
            pltpu.VMEM((CA, D), jnp.float32),
            pltpu.VMEM((W, D), jnp.float32),
            pltpu.VMEM((W, D), jnp.float32),
            pltpu.VMEM((N_SMALL + LANES,), jnp.int32),
        ] + [pltpu.SemaphoreType.DMA] * 6,
    )
    def k(h_hbm, pre_hbm, idx_hbm, out_hbm, ob0, ob1, hb0, hb1, idxv,
          sp0, sp1, sh0, sh1, sw0, sw1):
        cid = lax.axis_index("c")
        sid = lax.axis_index("s")
        wid = sid * 2 + cid
        b = wid // H
        hh = wid % H

        pltpu.sync_copy(idx_hbm.at[b], idxv.at[pl.ds(0, N_SMALL)])

        def searchsorted(t):
            def bs_body(step, lo):
                probe = lo + jnp.int32(2 ** (9 - step))
                v = idxv[pl.ds(probe - 1, LANES)][0]
                return jnp.where((probe <= N_SMALL) & (v < t), probe, lo)
            return lax.fori_loop(0, 10, bs_body, jnp.int32(0), unroll=True)
        js = [jnp.int32(0)] + [searchsorted(jnp.int32(t)) for t in THRESH]
        j0a = [(jnp.minimum(j, jnp.int32(N_SMALL - W)) // 8) * 8
               for j in js]

        obufs = (ob0, ob1)
        hbufs = (hb0, hb1)
        psems = (sp0, sp1)
        hsems = (sh0, sh1)
        wsems = (sw0, sw1)
        p_cps = [None, None]
        h_cps = [None, None]
        w_cps = [None, None]

        def start_reads(c):
            bi = c & 1
            n0, cc = CHUNKS[c]
            p_cps[bi] = pltpu.async_copy(
                pre_hbm.at[b, hh, pl.ds(n0, cc)],
                obufs[bi].at[pl.ds(0, cc)], psems[bi])
            h_cps[bi] = pltpu.async_copy(
                h_hbm.at[b, hh, pl.ds(j0a[c], W)], hbufs[bi], hsems[bi])

        start_reads(0)
        for c in range(len(CHUNKS)):
            bi = c & 1
            n0, cc = CHUNKS[c]
            p_cps[bi].wait()
            h_cps[bi].wait()
            if c + 1 < len(CHUNKS):
                ob = (c + 1) & 1
                if w_cps[ob] is not None:
                    w_cps[ob].wait()
                start_reads(c + 1)

            obuf = obufs[bi]
            hbuf = hbufs[bi]
            base = j0a[c]
            je = js[c + 1] if c + 1 < len(CHUNKS) else jnp.int32(N_SMALL)

            def place(j, carry):
                t = idxv[pl.ds(j, LANES)][0] - n0
                hr = j - base
                for l in range(D // LANES):
                    sl = pl.ds(l * LANES, LANES)
                    plsc.addupdate(obuf.at[t, sl], hbuf[hr, sl])
                return carry
            lax.fori_loop(js[c], je, place, 0, unroll=False)

            w_cps[bi] = pltpu.async_copy(
                obuf.at[pl.ds(0, cc)],
                out_hbm.at[b, hh, pl.ds(n0, cc)], wsems[bi])
        w_cps[0].wait()
        w_cps[1].wait()

    return k(h, pre_h, idx32)


def kernel(g, h, pre_h, idx):
    idx32 = idx.astype(jnp.int32)
    new_h = _sc_unpool(h, pre_h, idx32)
    g_out = _tc_copy(g.reshape(B * N, N), 1024).reshape(B, N, N)
    return (g_out, new_h)
